# Initial kernel scaffold; baseline (speedup 1.0000x reference)
import functools

import jax
import jax.numpy as jnp
from jax import lax
from jax.experimental import pallas as pl
from jax.experimental.pallas import tpu as pltpu
from jax.experimental.pallas import tpu_sc as plsc

N = 16384
K = 26
G = 100
W = K * G

NUM_CORES = 2
NUM_SUBCORES = 16
NW = NUM_CORES * NUM_SUBCORES
ROWS_PER_W = N // NW
C = 16
CHUNKS = ROWS_PER_W // C
L = 16


def _worker_body(x_hbm, zeros_hbm, out_hbm, xv, buf0, buf1, pos0, pos1,
                 sem0, sem1):
    wid = lax.axis_index("s") * NUM_CORES + lax.axis_index("c")
    base_row = wid * ROWS_PER_W

    pltpu.sync_copy(x_hbm.at[pl.ds(base_row, ROWS_PER_W), :], xv)
    pltpu.sync_copy(zeros_hbm, buf0)
    pltpu.sync_copy(zeros_hbm, buf1)

    ci = lax.iota(jnp.int32, L)
    base_lo = ci * G
    base_hi = (ci + 10) * G
    ones_v = jnp.full((L,), 1.0, jnp.float32)
    zero_v = jnp.zeros((L,), jnp.float32)
    one_i = jnp.full((L,), 1, jnp.int32)
    dummy_r = jnp.full((L,), C, jnp.int32)

    def fill(g, buf, posb):
        for r in range(C):
            rl = g * C + r
            r_splat = jnp.full((L,), r, jnp.int32)
            for half, base_c in ((0, base_lo), (1, base_hi)):
                v = xv[rl, pl.ds(10 * half, L)]
                valid = v > 0
                cv = base_c + jnp.maximum(v, one_i) - 1
                rv = jnp.where(valid, r_splat, dummy_r)
                plsc.store_scatter(buf, [rv, cv], ones_v)
                posb[pl.ds(r * 4 * L + 2 * half * L, L)] = rv
                posb[pl.ds(r * 4 * L + (2 * half + 1) * L, L)] = cv

    def clear(buf, posb):
        for r in range(C):
            for half in (0, 1):
                rv = posb[pl.ds(r * 4 * L + 2 * half * L, L)]
                cv = posb[pl.ds(r * 4 * L + (2 * half + 1) * L, L)]
                plsc.store_scatter(buf, [rv, cv], zero_v)

    def dma_out(g, buf, sem):
        row0 = base_row + g * C
        return pltpu.make_async_copy(
            buf.at[pl.ds(0, C), :], out_hbm.at[pl.ds(row0, C), :], sem)

    fill(0, buf0, pos0)
    dma_out(0, buf0, sem0).start()
    fill(1, buf1, pos1)
    dma_out(1, buf1, sem1).start()

    def body(gg, _):
        for b, buf, posb, sem in ((0, buf0, pos0, sem0),
                                  (1, buf1, pos1, sem1)):
            g = 2 * gg + b
            dma_out(g - 2, buf, sem).wait()
            clear(buf, posb)
            fill(g, buf, posb)
            dma_out(g, buf, sem).start()
        return 0

    lax.fori_loop(1, CHUNKS // 2, body, 0)

    dma_out(CHUNKS - 2, buf0, sem0).wait()
    dma_out(CHUNKS - 1, buf1, sem1).wait()


_sc_call = functools.partial(
    pl.kernel,
    out_type=jax.ShapeDtypeStruct((N, W), jnp.float32),
    mesh=plsc.VectorSubcoreMesh(core_axis_name="c", subcore_axis_name="s",
                                num_cores=NUM_CORES,
                                num_subcores=NUM_SUBCORES),
    scratch_types=[
        pltpu.VMEM((ROWS_PER_W, K), jnp.int32),
        pltpu.VMEM((C + 1, W), jnp.float32),
        pltpu.VMEM((C + 1, W), jnp.float32),
        pltpu.VMEM((C * 4 * L,), jnp.int32),
        pltpu.VMEM((C * 4 * L,), jnp.int32),
        pltpu.SemaphoreType.DMA,
        pltpu.SemaphoreType.DMA,
    ],
)(_worker_body)


def kernel(x):
    zeros = jnp.zeros((C + 1, W), jnp.float32)
    return _sc_call(x.astype(jnp.int32), zeros)


# speedup vs baseline: 1.2527x; 1.2527x over previous
import functools

import jax
import jax.numpy as jnp
from jax import lax
from jax.experimental import pallas as pl
from jax.experimental.pallas import tpu as pltpu
from jax.experimental.pallas import tpu_sc as plsc

N = 16384
K = 26
G = 100
W = K * G

NUM_CORES = 2
NUM_SUBCORES = 16
NW = NUM_CORES * NUM_SUBCORES
ROWS_PER_W = N // NW
C = 16
CHUNKS = ROWS_PER_W // C
L = 16


def _worker_body(x_hbm, zeros_hbm, out_hbm, xv, buf0, buf1, pos0, pos1,
                 sem0, sem1):
    wid = lax.axis_index("s") * NUM_CORES + lax.axis_index("c")
    base_row = wid * ROWS_PER_W

    pltpu.sync_copy(x_hbm.at[pl.ds(base_row, ROWS_PER_W), :], xv)
    pltpu.sync_copy(zeros_hbm, buf0)
    pltpu.sync_copy(zeros_hbm, buf1)

    ci = lax.iota(jnp.int32, L)
    base_lo = ci * G
    base_hi = (ci + 10) * G
    ones_v = jnp.full((L,), 1.0, jnp.float32)
    zero_v = jnp.zeros((L,), jnp.float32)
    one_i = jnp.full((L,), 1, jnp.int32)
    dummy_r = jnp.full((L,), C, jnp.int32)

    def fill(g, buf, posb):
        for r in range(C):
            rl = g * C + r
            r_splat = jnp.full((L,), r, jnp.int32)
            for half, base_c in ((0, base_lo), (1, base_hi)):
                v = xv[rl, pl.ds(10 * half, L)]
                valid = v > 0
                cv = base_c + jnp.maximum(v, one_i) - 1
                rv = jnp.where(valid, r_splat, dummy_r)
                plsc.store_scatter(buf, [rv, cv], ones_v)
                posb[pl.ds(r * 4 * L + 2 * half * L, L)] = rv
                posb[pl.ds(r * 4 * L + (2 * half + 1) * L, L)] = cv

    def clear(buf, posb):
        for r in range(C):
            for half in (0, 1):
                rv = posb[pl.ds(r * 4 * L + 2 * half * L, L)]
                cv = posb[pl.ds(r * 4 * L + (2 * half + 1) * L, L)]
                plsc.store_scatter(buf, [rv, cv], zero_v)

    def dma_out(g, buf, sem):
        row0 = base_row + g * C
        return pltpu.make_async_copy(
            buf.at[pl.ds(0, C), :], out_hbm.at[pl.ds(row0, C), :], sem)

    fill(0, buf0, pos0)
    dma_out(0, buf0, sem0).start()
    fill(1, buf1, pos1)
    dma_out(1, buf1, sem1).start()

    def body(gg, _):
        for b, buf, posb, sem in ((0, buf0, pos0, sem0),
                                  (1, buf1, pos1, sem1)):
            g = 2 * gg + b
            dma_out(g - 2, buf, sem).wait()
            clear(buf, posb)
            fill(g, buf, posb)
            dma_out(g, buf, sem).start()
        return 0

    lax.fori_loop(1, CHUNKS // 2, body, 0)

    dma_out(CHUNKS - 2, buf0, sem0).wait()
    dma_out(CHUNKS - 1, buf1, sem1).wait()


_sc_call = functools.partial(
    pl.kernel,
    out_type=jax.ShapeDtypeStruct((N, W), jnp.float32),
    mesh=plsc.VectorSubcoreMesh(core_axis_name="c", subcore_axis_name="s",
                                num_cores=NUM_CORES,
                                num_subcores=NUM_SUBCORES),
    scratch_types=[
        pltpu.VMEM((ROWS_PER_W, K), jnp.int32),
        pltpu.VMEM((C + 1, W), jnp.float32),
        pltpu.VMEM((C + 1, W), jnp.float32),
        pltpu.VMEM((C * 4 * L,), jnp.int32),
        pltpu.VMEM((C * 4 * L,), jnp.int32),
        pltpu.SemaphoreType.DMA,
        pltpu.SemaphoreType.DMA,
    ],
    compiler_params=pltpu.CompilerParams(use_tc_tiling_on_sc=False,
                                         needs_layout_passes=False),
)(_worker_body)


def kernel(x):
    zeros = jnp.zeros((C + 1, W), jnp.float32)
    return _sc_call(x.astype(jnp.int32), zeros)


# speedup vs baseline: 2.0347x; 1.6242x over previous
import functools

import jax
import jax.numpy as jnp
from jax import lax
from jax.experimental import pallas as pl
from jax.experimental.pallas import tpu as pltpu
from jax.experimental.pallas import tpu_sc as plsc

N = 16384
K = 26
G = 100
W = K * G

NUM_CORES = 2
NUM_SUBCORES = 16
NW = NUM_CORES * NUM_SUBCORES
ROWS_PER_W = N // NW
C = 16
CHUNKS = ROWS_PER_W // C
L = 16


def _worker_body(x_hbm, zeros_hbm, out_hbm, xv, buf0, buf1, pos0, pos1,
                 sem0, sem1):
    wid = lax.axis_index("s") * NUM_CORES + lax.axis_index("c")
    base_row = wid * ROWS_PER_W

    pltpu.sync_copy(x_hbm.at[pl.ds(base_row * K, ROWS_PER_W * K)], xv)
    pltpu.sync_copy(zeros_hbm, buf0)
    pltpu.sync_copy(zeros_hbm, buf1)

    ci = lax.iota(jnp.int32, L)
    base_lo = ci * G
    base_hi = (ci + 10) * G
    ones_v = jnp.full((L,), 1.0, jnp.float32)
    zero_v = jnp.zeros((L,), jnp.float32)
    one_i = jnp.full((L,), 1, jnp.int32)

    def fill(g, buf, posb):
        for r in range(C):
            rl = g * C + r
            r_splat = jnp.full((L,), r, jnp.int32)
            for half, base_c in ((0, base_lo), (1, base_hi)):
                v = xv[pl.ds(rl * K + 10 * half, L)]
                cv = base_c + jnp.maximum(v, one_i) - 1
                plsc.store_scatter(buf, [r_splat, cv], ones_v, mask=v > 0)
                posb[pl.ds((2 * r + half) * L, L)] = cv

    def clear(buf, posb):
        for r in range(C):
            r_splat = jnp.full((L,), r, jnp.int32)
            for half in (0, 1):
                cv = posb[pl.ds((2 * r + half) * L, L)]
                plsc.store_scatter(buf, [r_splat, cv], zero_v)

    def dma_out(g, buf, sem):
        row0 = base_row + g * C
        return pltpu.make_async_copy(
            buf, out_hbm.at[pl.ds(row0, C), :], sem)

    fill(0, buf0, pos0)
    dma_out(0, buf0, sem0).start()
    fill(1, buf1, pos1)
    dma_out(1, buf1, sem1).start()

    def body(gg, _):
        for b, buf, posb, sem in ((0, buf0, pos0, sem0),
                                  (1, buf1, pos1, sem1)):
            g = 2 * gg + b
            dma_out(g - 2, buf, sem).wait()
            clear(buf, posb)
            fill(g, buf, posb)
            dma_out(g, buf, sem).start()
        return 0

    lax.fori_loop(1, CHUNKS // 2, body, 0)

    dma_out(CHUNKS - 2, buf0, sem0).wait()
    dma_out(CHUNKS - 1, buf1, sem1).wait()


_sc_call = functools.partial(
    pl.kernel,
    out_type=jax.ShapeDtypeStruct((N, W), jnp.float32),
    mesh=plsc.VectorSubcoreMesh(core_axis_name="c", subcore_axis_name="s",
                                num_cores=NUM_CORES,
                                num_subcores=NUM_SUBCORES),
    scratch_types=[
        pltpu.VMEM((ROWS_PER_W * K,), jnp.int32),
        pltpu.VMEM((C, W), jnp.float32),
        pltpu.VMEM((C, W), jnp.float32),
        pltpu.VMEM((C * 2 * L,), jnp.int32),
        pltpu.VMEM((C * 2 * L,), jnp.int32),
        pltpu.SemaphoreType.DMA,
        pltpu.SemaphoreType.DMA,
    ],
    compiler_params=pltpu.CompilerParams(use_tc_tiling_on_sc=True,
                                         needs_layout_passes=False),
)(_worker_body)


def kernel(x):
    zeros = jnp.zeros((C, W), jnp.float32)
    return _sc_call(x.reshape(-1).astype(jnp.int32), zeros)
